# baseline (speedup 1.0000x reference)
import jax
import jax.numpy as jnp
from jax.experimental import pallas as pl
from jax.experimental.pallas import tpu as pltpu


def _fused_body(sup_ref, feat_ref, aggw_ref, catw_ref, out_ref, featb_ref):
    i = pl.program_id(0)
    bm = sup_ref.shape[0]
    d_in = feat_ref.shape[1]

    @pl.when(i == 0)
    def _():
        featb_ref[...] = feat_ref[...].astype(jnp.bfloat16)

    a = sup_ref[...].astype(jnp.bfloat16)
    feat_agg = jnp.dot(a, featb_ref[...],
                       preferred_element_type=jnp.float32)
    agg_out = jnp.dot(feat_agg, aggw_ref[...],
                      preferred_element_type=jnp.float32)
    featr = feat_ref[pl.ds(i * bm, bm), :]
    x = (jnp.dot(agg_out, catw_ref[:d_in, :],
                 preferred_element_type=jnp.float32)
         + jnp.dot(featr, catw_ref[d_in:, :],
                   preferred_element_type=jnp.float32))
    x = jnp.maximum(x, 0.0)
    n2 = jnp.sum(x * x, axis=1, keepdims=True)
    out_ref[...] = x / jnp.maximum(jnp.sqrt(n2), 1e-12)


def _pick_bm(n: int) -> int:
    for bm in (200, 80, 40, 16, 8):
        if n % bm == 0:
            return bm
    return 1


def kernel(feat, sup, agg_wei, cat_wei):
    n, d_in = feat.shape
    d_out = cat_wei.shape[1]
    bm = _pick_bm(n)
    return pl.pallas_call(
        _fused_body,
        grid=(n // bm,),
        in_specs=[
            pl.BlockSpec((bm, n), lambda i: (i, 0)),
            pl.BlockSpec((n, d_in), lambda i: (0, 0)),
            pl.BlockSpec((d_in, d_in), lambda i: (0, 0)),
            pl.BlockSpec((2 * d_in, d_out), lambda i: (0, 0)),
        ],
        out_specs=pl.BlockSpec((bm, d_out), lambda i: (i, 0)),
        out_shape=jax.ShapeDtypeStruct((n, d_out), jnp.float32),
        scratch_shapes=[pltpu.VMEM((n, d_in), jnp.bfloat16)],
        compiler_params=pltpu.CompilerParams(
            dimension_semantics=("arbitrary",),
            vmem_limit_bytes=100 * 1024 * 1024,
        ),
    )(sup, feat, agg_wei, cat_wei)


# speedup vs baseline: 1.0880x; 1.0880x over previous
import jax
import jax.numpy as jnp
from jax.experimental import pallas as pl
from jax.experimental.pallas import tpu as pltpu


def _fused_body(sup_ref, feat_ref, aggw_ref, catw_ref, out_ref, featb_ref):
    i = pl.program_id(0)
    bm = sup_ref.shape[0]
    d_in = feat_ref.shape[1]

    @pl.when(i == 0)
    def _():
        featb_ref[...] = feat_ref[...].astype(jnp.bfloat16)

    a = sup_ref[...].astype(jnp.bfloat16)
    feat_agg = jnp.dot(a, featb_ref[...],
                       preferred_element_type=jnp.float32)
    agg_out = jnp.dot(feat_agg, aggw_ref[...],
                      preferred_element_type=jnp.float32)
    featr = feat_ref[pl.ds(i * bm, bm), :]
    x = (jnp.dot(agg_out, catw_ref[:d_in, :],
                 preferred_element_type=jnp.float32)
         + jnp.dot(featr, catw_ref[d_in:, :],
                   preferred_element_type=jnp.float32))
    x = jnp.maximum(x, 0.0)
    n2 = jnp.sum(x * x, axis=1, keepdims=True)
    out_ref[...] = x / jnp.maximum(jnp.sqrt(n2), 1e-12)


def _pick_bm(n: int) -> int:
    for bm in (400, 200, 80, 40, 16, 8):
        if n % bm == 0:
            return bm
    return 1


def kernel(feat, sup, agg_wei, cat_wei):
    n, d_in = feat.shape
    d_out = cat_wei.shape[1]
    bm = _pick_bm(n)
    return pl.pallas_call(
        _fused_body,
        grid=(n // bm,),
        in_specs=[
            pl.BlockSpec((bm, n), lambda i: (i, 0)),
            pl.BlockSpec((n, d_in), lambda i: (0, 0)),
            pl.BlockSpec((d_in, d_in), lambda i: (0, 0)),
            pl.BlockSpec((2 * d_in, d_out), lambda i: (0, 0)),
        ],
        out_specs=pl.BlockSpec((bm, d_out), lambda i: (i, 0)),
        out_shape=jax.ShapeDtypeStruct((n, d_out), jnp.float32),
        scratch_shapes=[pltpu.VMEM((n, d_in), jnp.bfloat16)],
        compiler_params=pltpu.CompilerParams(
            dimension_semantics=("arbitrary",),
            vmem_limit_bytes=100 * 1024 * 1024,
        ),
    )(sup, feat, agg_wei, cat_wei)
